# Initial kernel scaffold; baseline (speedup 1.0000x reference)
import jax
import jax.numpy as jnp
from jax.experimental import pallas as pl


def kernel(x, membership, W_merge, b_merge):
    raise NotImplementedError("write your pallas kernel here")


# speedup vs baseline: 2.5612x; 2.5612x over previous
import functools

import jax
import jax.numpy as jnp
from jax import lax
from jax.experimental import pallas as pl
from jax.experimental.pallas import tpu as pltpu
from jax.experimental.pallas import tpu_sc as plsc

B_SEG = 2048
NC, NS = 2, 16
NW = NC * NS
SEG_PER_W = B_SEG // NW
CHUNK = 256
LANES = 16
NEG_INF = float("-inf")


def _sc_segment_reduce(x, m32, starts):
    N, D = x.shape
    nvec = D // LANES
    mesh = plsc.VectorSubcoreMesh(core_axis_name="c", subcore_axis_name="s")

    @functools.partial(
        pl.kernel,
        out_type=(
            jax.ShapeDtypeStruct((B_SEG, D), jnp.float32),
            jax.ShapeDtypeStruct((B_SEG, D), jnp.float32),
        ),
        mesh=mesh,
        scratch_types=[
            pltpu.VMEM((CHUNK, D), jnp.float32),
            pltpu.VMEM((CHUNK + LANES,), jnp.int32),
            pltpu.VMEM((SEG_PER_W, D), jnp.float32),
            pltpu.VMEM((SEG_PER_W, D), jnp.float32),
            pltpu.VMEM((48,), jnp.int32),
        ],
    )
    def seg_kernel(x_hbm, m_hbm, starts_hbm, sum_hbm, max_hbm,
                   xbuf, mbuf, acc_s, acc_m, bnd):
        w = lax.axis_index("s") * NC + lax.axis_index("c")
        pltpu.sync_copy(starts_hbm, bnd)
        bv = bnd[pl.ds(w, LANES)]
        r0 = bv[0]
        r1 = bv[1]
        seg_lo = w * SEG_PER_W

        zeros = jnp.zeros((LANES,), jnp.float32)
        ninf = jnp.full((LANES,), NEG_INF, jnp.float32)

        def init_body(i, _):
            s = i // nvec
            j = i % nvec
            acc_s[s, pl.ds(j * LANES, LANES)] = zeros
            acc_m[s, pl.ds(j * LANES, LANES)] = ninf
            return 0

        lax.fori_loop(0, SEG_PER_W * nvec, init_body, 0)

        a0 = (r0 // 8) * 8
        nchunks = (r1 - a0 + CHUNK - 1) // CHUNK

        def chunk_body(k, _):
            s_un = a0 + k * CHUNK
            s = jnp.minimum(s_un, N - CHUNK)
            pltpu.sync_copy(x_hbm.at[pl.ds(s, CHUNK)], xbuf)
            pltpu.sync_copy(m_hbm.at[pl.ds(s, CHUNK)], mbuf.at[pl.ds(0, CHUNK)])
            lo = jnp.maximum(r0, s_un) - s
            hi = jnp.minimum(r1, s_un + CHUNK) - s

            def row_body(i, _):
                seg = mbuf[pl.ds(i, LANES)][0] - seg_lo
                for j in range(nvec):
                    sl = pl.ds(j * LANES, LANES)
                    v = xbuf[i, sl]
                    acc_s[seg, sl] = acc_s[seg, sl] + v
                    acc_m[seg, sl] = jnp.maximum(acc_m[seg, sl], v)
                return 0

            lax.fori_loop(lo, hi, row_body, 0)
            return 0

        lax.fori_loop(0, nchunks, chunk_body, 0)

        pltpu.sync_copy(acc_s, sum_hbm.at[pl.ds(seg_lo, SEG_PER_W)])
        pltpu.sync_copy(acc_m, max_hbm.at[pl.ds(seg_lo, SEG_PER_W)])

    return seg_kernel(x, m32, starts)


def _tc_merge(seg_max, seg_sum, W_merge, b_merge):
    B, D = seg_max.shape

    def body(mx_ref, sm_ref, w_ref, b_ref, o_ref):
        mx = mx_ref[...]
        mx = jnp.where(jnp.isfinite(mx), mx, 0.0)
        acc = jnp.dot(mx, w_ref[0:D, :], preferred_element_type=jnp.float32)
        acc = acc + jnp.dot(sm_ref[...], w_ref[D:2 * D, :],
                            preferred_element_type=jnp.float32)
        o_ref[...] = acc + b_ref[...]

    return pl.pallas_call(
        body,
        out_shape=jax.ShapeDtypeStruct((B, W_merge.shape[1]), jnp.float32),
    )(seg_max, seg_sum, W_merge, b_merge)


def kernel(x, membership, W_merge, b_merge):
    m32 = membership.astype(jnp.int32)
    edges = jnp.arange(0, B_SEG + 1, SEG_PER_W, dtype=jnp.int32)
    starts = jnp.searchsorted(m32, edges, side="left").astype(jnp.int32)
    starts = jnp.pad(starts, (0, 15))
    seg_sum, seg_max = _sc_segment_reduce(x, m32, starts)
    return _tc_merge(seg_max, seg_sum, W_merge, jnp.reshape(b_merge, (1, -1)))
